# baseline (speedup 1.0000x reference)
import jax
import jax.numpy as jnp
from jax.experimental import pallas as pl
from jax.experimental.pallas import tpu as pltpu

_TOKENS = 2048
_HIDDEN = 1024
_INTER = 4096
_TM = 1024
_BI = 512
_NI = _INTER // _BI
_DIMS = (((1,), (1,)), ((), ()))


def _ffn_body(x_ref, w1_ref, w3_ref, w2_ref, y_ref):
    j = pl.program_id(1)
    x = x_ref[...]
    h1 = jax.lax.dot_general(x, w1_ref[...], _DIMS, preferred_element_type=jnp.float32)
    h3 = jax.lax.dot_general(x, w3_ref[...], _DIMS, preferred_element_type=jnp.float32)
    g = jax.nn.silu(h1) * h3
    contrib = jax.lax.dot_general(g, w2_ref[...], _DIMS, preferred_element_type=jnp.float32)

    @pl.when(j == 0)
    def _init():
        y_ref[...] = contrib

    @pl.when(j > 0)
    def _acc():
        y_ref[...] += contrib


def kernel(hidden_states, gate_w, w1, w2, w3):
    del gate_w
    grid = (_TOKENS // _TM, _NI)
    return pl.pallas_call(
        _ffn_body,
        grid=grid,
        in_specs=[
            pl.BlockSpec((_TM, _HIDDEN), lambda t, j: (t, 0)),
            pl.BlockSpec((_BI, _HIDDEN), lambda t, j: (j, 0)),
            pl.BlockSpec((_BI, _HIDDEN), lambda t, j: (j, 0)),
            pl.BlockSpec((_HIDDEN, _BI), lambda t, j: (0, j)),
        ],
        out_specs=pl.BlockSpec((_TM, _HIDDEN), lambda t, j: (t, 0)),
        out_shape=jax.ShapeDtypeStruct((_TOKENS, _HIDDEN), jnp.float32),
        compiler_params=pltpu.CompilerParams(
            dimension_semantics=("arbitrary", "arbitrary"),
        ),
    )(hidden_states, w1[0], w3[0], w2[0])


# speedup vs baseline: 1.0661x; 1.0661x over previous
import jax
import jax.numpy as jnp
from jax.experimental import pallas as pl
from jax.experimental.pallas import tpu as pltpu

_TOKENS = 2048
_HIDDEN = 1024
_INTER = 4096
_TM = 1024
_NT = _TOKENS // _TM
_BI = 512
_NI = _INTER // _BI
_DIMS = (((1,), (1,)), ((), ()))


def _ffn_body(x_hbm, w1_hbm, w3_hbm, w2_hbm, y_hbm,
              xv, w1v, w3v, w2v, yv,
              x_sem, y_sem, w1_sem, w3_sem, w2_sem):
    def w_copies(j, buf):
        row = pl.ds(j * _BI, _BI)
        return (
            pltpu.make_async_copy(w1_hbm.at[row, :], w1v.at[buf], w1_sem.at[buf]),
            pltpu.make_async_copy(w3_hbm.at[row, :], w3v.at[buf], w3_sem.at[buf]),
            pltpu.make_async_copy(w2_hbm.at[:, row], w2v.at[buf], w2_sem.at[buf]),
        )

    x_copy = pltpu.make_async_copy(x_hbm, xv, x_sem)
    x_copy.start()
    for c in w_copies(0, 0):
        c.start()

    for j in range(_NI):
        cur = j % 2
        if j + 1 < _NI:
            for c in w_copies(j + 1, (j + 1) % 2):
                c.start()
        for c in w_copies(j, cur):
            c.wait()
        if j == 0:
            x_copy.wait()
        w1b = w1v[cur]
        w3b = w3v[cur]
        w2b = w2v[cur]
        for t in range(_NT):
            rows = slice(t * _TM, (t + 1) * _TM)
            xt = xv[rows, :]
            h1 = jax.lax.dot_general(xt, w1b, _DIMS, preferred_element_type=jnp.float32)
            h3 = jax.lax.dot_general(xt, w3b, _DIMS, preferred_element_type=jnp.float32)
            g = jax.nn.silu(h1) * h3
            contrib = jax.lax.dot_general(g, w2b, _DIMS, preferred_element_type=jnp.float32)
            if j == 0:
                yv[rows, :] = contrib
            else:
                yv[rows, :] += contrib

    y_copy = pltpu.make_async_copy(yv, y_hbm, y_sem)
    y_copy.start()
    y_copy.wait()


def kernel(hidden_states, gate_w, w1, w2, w3):
    del gate_w
    return pl.pallas_call(
        _ffn_body,
        in_specs=[
            pl.BlockSpec(memory_space=pl.ANY),
            pl.BlockSpec(memory_space=pl.ANY),
            pl.BlockSpec(memory_space=pl.ANY),
            pl.BlockSpec(memory_space=pl.ANY),
        ],
        out_specs=pl.BlockSpec(memory_space=pl.ANY),
        out_shape=jax.ShapeDtypeStruct((_TOKENS, _HIDDEN), jnp.float32),
        scratch_shapes=[
            pltpu.VMEM((_TOKENS, _HIDDEN), jnp.float32),
            pltpu.VMEM((2, _BI, _HIDDEN), jnp.float32),
            pltpu.VMEM((2, _BI, _HIDDEN), jnp.float32),
            pltpu.VMEM((2, _HIDDEN, _BI), jnp.float32),
            pltpu.VMEM((_TOKENS, _HIDDEN), jnp.float32),
            pltpu.SemaphoreType.DMA,
            pltpu.SemaphoreType.DMA,
            pltpu.SemaphoreType.DMA((2,)),
            pltpu.SemaphoreType.DMA((2,)),
            pltpu.SemaphoreType.DMA((2,)),
        ],
    )(hidden_states, w1[0], w3[0], w2[0])


# speedup vs baseline: 1.1130x; 1.0440x over previous
import jax
import jax.numpy as jnp
from jax.experimental import pallas as pl
from jax.experimental.pallas import tpu as pltpu

_TOKENS = 2048
_HIDDEN = 1024
_INTER = 4096
_TM = 1024
_NT = _TOKENS // _TM
_BI = 512
_NI = _INTER // _BI
_DIMS = (((1,), (1,)), ((), ()))


def _ffn_body(x_hbm, w1_hbm, w3_hbm, w2_hbm, y_hbm,
              xv, w1v, w3v, w2v, yv,
              x_sem, y_sem, w1_sem, w3_sem, w2_sem):
    def w_copies(j, buf):
        row = pl.ds(j * _BI, _BI)
        return (
            pltpu.make_async_copy(w1_hbm.at[row, :], w1v.at[buf], w1_sem.at[buf]),
            pltpu.make_async_copy(w3_hbm.at[row, :], w3v.at[buf], w3_sem.at[buf]),
            pltpu.make_async_copy(w2_hbm.at[:, row], w2v.at[buf], w2_sem.at[buf]),
        )

    def half(ref, t):
        return ref.at[pl.ds(t * _TM, _TM), :]

    x_copies = [
        pltpu.make_async_copy(half(x_hbm, t), half(xv, t), x_sem.at[t])
        for t in range(_NT)
    ]
    y_copies = [
        pltpu.make_async_copy(half(yv, t), half(y_hbm, t), y_sem.at[t])
        for t in range(_NT)
    ]
    x_copies[0].start()
    first_w = w_copies(0, 0)
    for c in first_w:
        c.start()
    x_copies[1].start()

    pending_w = first_w
    for j in range(_NI):
        cur = j % 2
        if j + 1 < _NI:
            next_w = w_copies(j + 1, (j + 1) % 2)
            for c in next_w:
                c.start()
        w1c, w3c, w2c = pending_w
        if j + 1 < _NI:
            pending_w = next_w
        if j == 0:
            x_copies[0].wait()
        w1c.wait()
        w3c.wait()
        w2c.wait()
        w1b = w1v[cur]
        w3b = w3v[cur]
        w2b = w2v[cur]
        for t in range(_NT):
            if j == 0 and t == 1:
                x_copies[1].wait()
            rows = slice(t * _TM, (t + 1) * _TM)
            xt = xv[rows, :]
            h1 = jax.lax.dot_general(xt, w1b, _DIMS, preferred_element_type=jnp.float32)
            h3 = jax.lax.dot_general(xt, w3b, _DIMS, preferred_element_type=jnp.float32)
            g = jax.nn.silu(h1) * h3
            contrib = jax.lax.dot_general(g, w2b, _DIMS, preferred_element_type=jnp.float32)
            if j == 0:
                yv[rows, :] = contrib
            else:
                yv[rows, :] += contrib
            if j == _NI - 1:
                y_copies[t].start()

    for t in range(_NT):
        y_copies[t].wait()


def kernel(hidden_states, gate_w, w1, w2, w3):
    del gate_w
    return pl.pallas_call(
        _ffn_body,
        in_specs=[
            pl.BlockSpec(memory_space=pl.ANY),
            pl.BlockSpec(memory_space=pl.ANY),
            pl.BlockSpec(memory_space=pl.ANY),
            pl.BlockSpec(memory_space=pl.ANY),
        ],
        out_specs=pl.BlockSpec(memory_space=pl.ANY),
        out_shape=jax.ShapeDtypeStruct((_TOKENS, _HIDDEN), jnp.float32),
        scratch_shapes=[
            pltpu.VMEM((_TOKENS, _HIDDEN), jnp.float32),
            pltpu.VMEM((2, _BI, _HIDDEN), jnp.float32),
            pltpu.VMEM((2, _BI, _HIDDEN), jnp.float32),
            pltpu.VMEM((2, _HIDDEN, _BI), jnp.float32),
            pltpu.VMEM((_TOKENS, _HIDDEN), jnp.float32),
            pltpu.SemaphoreType.DMA((2,)),
            pltpu.SemaphoreType.DMA((2,)),
            pltpu.SemaphoreType.DMA((2,)),
            pltpu.SemaphoreType.DMA((2,)),
            pltpu.SemaphoreType.DMA((2,)),
        ],
    )(hidden_states, w1[0], w3[0], w2[0])
